# baseline (speedup 1.0000x reference)
import functools

import jax
import jax.numpy as jnp
from jax import lax
from jax.experimental import pallas as pl
from jax.experimental.pallas import tpu as pltpu

NE = 8
TOPK = 2
TM = 256
RT = 256
LANES = 128


def _router_body(x_ref, wr_ref, e_ref, w_ref):
    xt = x_ref[...]
    logits = jnp.dot(xt, wr_ref[...], preferred_element_type=jnp.float32)
    lane = lax.broadcasted_iota(jnp.int32, logits.shape, 1)
    logits = jnp.where(lane < NE, logits, -1e30)
    m1 = jnp.max(logits, axis=1, keepdims=True)
    a1 = jnp.min(jnp.where(logits >= m1, lane, LANES), axis=1)
    masked = jnp.where(lane == a1[:, None], -1e30, logits)
    m2 = jnp.max(masked, axis=1, keepdims=True)
    a2 = jnp.min(jnp.where(masked >= m2, lane, LANES), axis=1)
    w1 = 1.0 / (1.0 + jnp.exp(m2[:, 0] - m1[:, 0]))
    e_ref[0, :] = a1
    e_ref[1, :] = a2
    w_ref[0, :] = w1
    w_ref[1, :] = 1.0 - w1


def _gelu(h):
    c = 0.7978845608028654
    return 0.5 * h * (1.0 + jnp.tanh(c * (h + 0.044715 * h * h * h)))


def _ffn_body(te_ref, xs_ref, w1_ref, b1_ref, w2_ref, b2_ref, ys_ref):
    xt = xs_ref[...].astype(jnp.bfloat16)
    h = jnp.dot(xt, w1_ref[0], preferred_element_type=jnp.float32) + b1_ref[0]
    h = _gelu(h).astype(jnp.bfloat16)
    ys_ref[...] = jnp.dot(h, w2_ref[0], preferred_element_type=jnp.float32) + b2_ref[0]


def _plan(e_flat, N, NP, NT):
    oh = (e_flat[:, None] == jnp.arange(NE)).astype(jnp.int32)
    csum = jnp.cumsum(oh, axis=0)
    rank = jnp.take_along_axis(csum, e_flat[:, None], axis=1)[:, 0] - 1
    cnt = csum[-1]
    padded = ((cnt + TM - 1) // TM) * TM
    ends = jnp.cumsum(padded)
    starts = ends - padded
    dest = starts[e_flat] + rank
    tok = jnp.concatenate([jnp.arange(N, dtype=jnp.int32)] * TOPK)
    src_idx = jnp.zeros((NP,), jnp.int32).at[dest].set(tok)
    d1, d2 = dest[:N], dest[N:]
    tile_e = jnp.clip(
        jnp.searchsorted(ends, jnp.arange(NT) * TM, side="right"), 0, NE - 1
    ).astype(jnp.int32)
    return src_idx, d1, d2, tile_e


def kernel(x, Wr, W1, b1, W2, b2):
    Bb, Tt, D = x.shape
    N = Bb * Tt
    DE = W1.shape[2]
    NT = (TOPK * N + NE * TM) // TM
    NP = NT * TM
    x_flat = x.reshape(N, D)

    wrp = jnp.zeros((D, LANES), jnp.float32).at[:, :NE].set(Wr)
    eidx, wgt = pl.pallas_call(
        _router_body,
        grid=(N // RT,),
        in_specs=[
            pl.BlockSpec((RT, D), lambda i: (i, 0)),
            pl.BlockSpec((D, LANES), lambda i: (0, 0)),
        ],
        out_specs=[
            pl.BlockSpec((TOPK, RT), lambda i: (0, i)),
            pl.BlockSpec((TOPK, RT), lambda i: (0, i)),
        ],
        out_shape=[
            jax.ShapeDtypeStruct((TOPK, N), jnp.int32),
            jax.ShapeDtypeStruct((TOPK, N), jnp.float32),
        ],
    )(x_flat, wrp)

    src_idx, d1, d2, tile_e = _plan(eidx.reshape(-1), N, NP, NT)

    xs = x_flat[src_idx]

    b1r = b1.reshape(NE, 1, DE)
    b2r = b2.reshape(NE, 1, D)
    ys = pl.pallas_call(
        _ffn_body,
        grid_spec=pltpu.PrefetchScalarGridSpec(
            num_scalar_prefetch=1,
            grid=(NT,),
            in_specs=[
                pl.BlockSpec((TM, D), lambda i, te: (i, 0)),
                pl.BlockSpec((1, D, DE), lambda i, te: (te[i], 0, 0)),
                pl.BlockSpec((1, 1, DE), lambda i, te: (te[i], 0, 0)),
                pl.BlockSpec((1, DE, D), lambda i, te: (te[i], 0, 0)),
                pl.BlockSpec((1, 1, D), lambda i, te: (te[i], 0, 0)),
            ],
            out_specs=pl.BlockSpec((TM, D), lambda i, te: (i, 0)),
        ),
        out_shape=jax.ShapeDtypeStruct((NP, D), jnp.float32),
        compiler_params=pltpu.CompilerParams(
            dimension_semantics=("arbitrary",),
        ),
    )(tile_e, xs, W1.astype(jnp.bfloat16), b1r, W2.astype(jnp.bfloat16), b2r)

    out = wgt[0][:, None] * ys[d1] + wgt[1][:, None] * ys[d2]
    return (out.reshape(Bb, Tt, D), jnp.float32(0.0))


# speedup vs baseline: 1.4013x; 1.4013x over previous
import functools

import jax
import jax.numpy as jnp
from jax import lax
from jax.experimental import pallas as pl
from jax.experimental.pallas import tpu as pltpu
from jax.experimental.pallas import tpu_sc as plsc

NE = 8
TOPK = 2
TM = 256
RT = 256
LANES = 128
NC, NS, NW = 2, 16, 32
CH = 256
GB = 64
TPW = 128
CB = 32


def _wid():
    return lax.axis_index("s") * NC + lax.axis_index("c")


_GDN = lax.GatherDimensionNumbers(
    offset_dims=(), collapsed_slice_dims=(0,), start_index_map=(0,))


def _splat(vec, lane):
    idx = jnp.full((16, 1), lane, jnp.int32)
    return lax.gather(vec, idx, _GDN, (1,),
                      mode=lax.GatherScatterMode.PROMISE_IN_BOUNDS)


def _shift_fwd(v, k, lane):
    idx = jnp.maximum(lane - k, 0)[:, None]
    g = lax.gather(v, idx, _GDN, (1,),
                   mode=lax.GatherScatterMode.PROMISE_IN_BOUNDS)
    return jnp.where(lane >= k, g, 0)


def _prefix_sum_incl(v, lane):
    s = v
    for k in (1, 2, 4, 8):
        s = s + _shift_fwd(s, k, lane)
    return s


def _router_body(x_ref, wr_ref, e_ref, w_ref):
    xt = x_ref[...]
    logits = jnp.dot(xt, wr_ref[...], preferred_element_type=jnp.float32)
    lane = lax.broadcasted_iota(jnp.int32, logits.shape, 1)
    logits = jnp.where(lane < NE, logits, -1e30)
    m1 = jnp.max(logits, axis=1, keepdims=True)
    a1 = jnp.min(jnp.where(logits >= m1, lane, LANES), axis=1)
    masked = jnp.where(lane == a1[:, None], -1e30, logits)
    m2 = jnp.max(masked, axis=1, keepdims=True)
    a2 = jnp.min(jnp.where(masked >= m2, lane, LANES), axis=1)
    w1 = 1.0 / (1.0 + jnp.exp(m2[:, 0] - m1[:, 0]))
    e_ref[0, :] = a1
    e_ref[1, :] = a2
    w_ref[0, :] = w1
    w_ref[1, :] = 1.0 - w1


def _gelu(h):
    c = 0.7978845608028654
    return 0.5 * h * (1.0 + jnp.tanh(c * (h + 0.044715 * h * h * h)))


def _ffn_body(te_ref, xs_ref, w1_ref, b1_ref, w2_ref, b2_ref, ys_ref):
    xt = xs_ref[...]
    h = jnp.dot(xt, w1_ref[0], preferred_element_type=jnp.float32) + b1_ref[0]
    h = _gelu(h)
    ys_ref[...] = jnp.dot(h, w2_ref[0], preferred_element_type=jnp.float32) + b2_ref[0]


def _plan_count_body(e_hbm, lr_hbm, cnt_hbm, e_v, lr_v, cnt_v):
    w = _wid()
    base = w * CH
    pltpu.sync_copy(e_hbm.at[pl.ds(base, CH)], e_v)
    lane = lax.iota(jnp.int32, 16)
    carries = [jnp.zeros((16,), jnp.int32) for _ in range(NE)]
    for v in range(CH // 16):
        ev = e_v[pl.ds(v * 16, 16)]
        lr = jnp.zeros((16,), jnp.int32)
        for e0 in range(NE):
            m = ev == e0
            incl = _prefix_sum_incl(jnp.where(m, 1, 0), lane)
            lr = jnp.where(m, carries[e0] + incl - 1, lr)
            carries[e0] = carries[e0] + _splat(incl, 15)
        lr_v[pl.ds(v * 16, 16)] = lr
    cnt = jnp.zeros((16,), jnp.int32)
    for e0 in range(NE):
        cnt = jnp.where(lane == e0, carries[e0], cnt)
    cnt_v[...] = cnt
    pltpu.sync_copy(lr_v, lr_hbm.at[pl.ds(base, CH)])
    pltpu.sync_copy(cnt_v, cnt_hbm.at[w])


def _plan_apply_body(N, e_hbm, lr_hbm, cnt_hbm, x_hbm,
                     xs_hbm, d_hbm, te_hbm,
                     e_v, lr_v, cnt_v, dest_v, tok_v, dlin_v, te_v, rbuf,
                     gsem, ssem):
    w = _wid()
    base = w * CH
    lane = lax.iota(jnp.int32, 16)

    pltpu.sync_copy(cnt_hbm, cnt_v)
    total = jnp.zeros((16,), jnp.int32)
    pre = jnp.zeros((16,), jnp.int32)
    for wp in range(NW):
        row = cnt_v[wp]
        coef = jnp.full((16,), (w > wp).astype(jnp.int32))
        pre = pre + row * coef
        total = total + row
    padded = ((total + (TM - 1)) >> 8) << 8
    ends = _prefix_sum_incl(padded, lane)
    starts = ends - padded
    basev = starts + pre

    pltpu.sync_copy(e_hbm.at[pl.ds(base, CH)], e_v)
    pltpu.sync_copy(lr_hbm.at[pl.ds(base, CH)], lr_v)
    soff = (w >= NW // TOPK).astype(jnp.int32) * N
    for v in range(CH // 16):
        ev = e_v[pl.ds(v * 16, 16)]
        dest = lr_v[pl.ds(v * 16, 16)]
        for e0 in range(NE):
            dest = dest + jnp.where(ev == e0, _splat(basev, e0), 0)
        tok = jnp.full((16,), base - soff + v * 16, jnp.int32) + lane
        dest_v[v // 4, pl.ds((v % 4) * 16, 16)] = dest
        tok_v[v // 4, pl.ds((v % 4) * 16, 16)] = tok
        dlin_v[pl.ds(v * 16, 16)] = dest
    pltpu.sync_copy(dlin_v, d_hbm.at[pl.ds(base, CH)])

    @pl.when(w == 0)
    def _():
        for v in range(3):
            tstart = (jnp.full((16,), v * 16, jnp.int32) + lane) * TM
            te = jnp.zeros((16,), jnp.int32)
            for e0 in range(NE):
                te = te + jnp.where(tstart >= _splat(ends, e0), 1, 0)
            te_v[pl.ds(v * 16, 16)] = te - (te >> 3)
        pltpu.sync_copy(te_v, te_hbm)

    for c in range(CH // GB):
        pltpu.async_copy(x_hbm.at[tok_v.at[c]], rbuf, gsem).wait()
        pltpu.async_copy(rbuf, xs_hbm.at[dest_v.at[c]], ssem).wait()


def _combine_body(N, ys_hbm, d_hbm, wgt_hbm, out_hbm,
                  d1_v, d2_v, w1_v, w2_v, b1_v, b2_v, s1, s2):
    w = _wid()
    tb = w * TPW
    pltpu.sync_copy(d_hbm.at[pl.ds(tb, TPW)], d1_v)
    pltpu.sync_copy(d_hbm.at[pl.ds(N + tb, TPW)], d2_v)
    pltpu.sync_copy(wgt_hbm.at[0, pl.ds(tb, TPW)], w1_v)
    pltpu.sync_copy(wgt_hbm.at[1, pl.ds(tb, TPW)], w2_v)
    for c in range(TPW // CB):
        cp1 = pltpu.async_copy(ys_hbm.at[d1_v.at[pl.ds(c * CB, CB)]], b1_v, s1)
        cp2 = pltpu.async_copy(ys_hbm.at[d2_v.at[pl.ds(c * CB, CB)]], b2_v, s2)
        cp1.wait()
        cp2.wait()

        def row_body(r, _):
            i = c * CB + r
            vbase = (i >> 4) << 4
            idx = jnp.full((16, 1), i - vbase, jnp.int32)
            w1r = lax.gather(w1_v[pl.ds(vbase, 16)], idx, _GDN, (1,),
                             mode=lax.GatherScatterMode.PROMISE_IN_BOUNDS)
            w2r = lax.gather(w2_v[pl.ds(vbase, 16)], idx, _GDN, (1,),
                             mode=lax.GatherScatterMode.PROMISE_IN_BOUNDS)

            def col_body(cc, _):
                a = b1_v[r, pl.ds(cc * 16, 16)]
                b = b2_v[r, pl.ds(cc * 16, 16)]
                b1_v[r, pl.ds(cc * 16, 16)] = w1r * a + w2r * b
                return 0

            lax.fori_loop(0, b1_v.shape[1] // 16, col_body, 0)
            return 0

        lax.fori_loop(0, CB, row_body, 0)
        pltpu.sync_copy(b1_v, out_hbm.at[pl.ds(tb + c * CB, CB)])


def kernel(x, Wr, W1, b1, W2, b2):
    Bb, Tt, D = x.shape
    N = Bb * Tt
    DE = W1.shape[2]
    NA = TOPK * N
    NT = (NA + NE * TM) // TM
    NP = NT * TM
    NTP = ((NT + 15) // 16) * 16
    x_flat = x.reshape(N, D)

    wrp = jnp.zeros((D, LANES), jnp.float32).at[:, :NE].set(Wr)
    eidx, wgt = pl.pallas_call(
        _router_body,
        grid=(N // RT,),
        in_specs=[
            pl.BlockSpec((RT, D), lambda i: (i, 0)),
            pl.BlockSpec((D, LANES), lambda i: (0, 0)),
        ],
        out_specs=[
            pl.BlockSpec((TOPK, RT), lambda i: (0, i)),
            pl.BlockSpec((TOPK, RT), lambda i: (0, i)),
        ],
        out_shape=[
            jax.ShapeDtypeStruct((TOPK, N), jnp.int32),
            jax.ShapeDtypeStruct((TOPK, N), jnp.float32),
        ],
    )(x_flat, wrp)
    e_flat = eidx.reshape(NA)

    mesh = plsc.VectorSubcoreMesh(core_axis_name="c", subcore_axis_name="s")

    lr, cnt = pl.kernel(
        _plan_count_body,
        out_type=[
            jax.ShapeDtypeStruct((NA,), jnp.int32),
            jax.ShapeDtypeStruct((NW, 16), jnp.int32),
        ],
        mesh=mesh,
        scratch_types=[
            pltpu.VMEM((CH,), jnp.int32),
            pltpu.VMEM((CH,), jnp.int32),
            pltpu.VMEM((16,), jnp.int32),
        ],
        name="moe_plan_count",
    )(e_flat)

    xs, d_all, tile_e = pl.kernel(
        functools.partial(_plan_apply_body, N),
        out_type=[
            jax.ShapeDtypeStruct((NP, D), jnp.float32),
            jax.ShapeDtypeStruct((NA,), jnp.int32),
            jax.ShapeDtypeStruct((NTP,), jnp.int32),
        ],
        mesh=mesh,
        scratch_types=[
            pltpu.VMEM((CH,), jnp.int32),
            pltpu.VMEM((CH,), jnp.int32),
            pltpu.VMEM((NW, 16), jnp.int32),
            pltpu.VMEM((CH // GB, GB), jnp.int32),
            pltpu.VMEM((CH // GB, GB), jnp.int32),
            pltpu.VMEM((CH,), jnp.int32),
            pltpu.VMEM((NTP,), jnp.int32),
            pltpu.VMEM((GB, D), jnp.float32),
            pltpu.SemaphoreType.DMA,
            pltpu.SemaphoreType.DMA,
        ],
        name="moe_plan_apply",
    )(e_flat, lr, cnt, x_flat)

    b1r = b1.reshape(NE, 1, DE)
    b2r = b2.reshape(NE, 1, D)
    ys = pl.pallas_call(
        _ffn_body,
        grid_spec=pltpu.PrefetchScalarGridSpec(
            num_scalar_prefetch=1,
            grid=(NT,),
            in_specs=[
                pl.BlockSpec((TM, D), lambda i, te: (i, 0)),
                pl.BlockSpec((1, D, DE), lambda i, te: (te[i], 0, 0)),
                pl.BlockSpec((1, 1, DE), lambda i, te: (te[i], 0, 0)),
                pl.BlockSpec((1, DE, D), lambda i, te: (te[i], 0, 0)),
                pl.BlockSpec((1, 1, D), lambda i, te: (te[i], 0, 0)),
            ],
            out_specs=pl.BlockSpec((TM, D), lambda i, te: (i, 0)),
        ),
        out_shape=jax.ShapeDtypeStruct((NP, D), jnp.float32),
        compiler_params=pltpu.CompilerParams(
            dimension_semantics=("arbitrary",),
        ),
    )(tile_e[:NT], xs, W1, b1r, W2, b2r)

    out = pl.kernel(
        functools.partial(_combine_body, N),
        out_type=jax.ShapeDtypeStruct((N, D), jnp.float32),
        mesh=mesh,
        scratch_types=[
            pltpu.VMEM((TPW,), jnp.int32),
            pltpu.VMEM((TPW,), jnp.int32),
            pltpu.VMEM((TPW,), jnp.float32),
            pltpu.VMEM((TPW,), jnp.float32),
            pltpu.VMEM((CB, D), jnp.float32),
            pltpu.VMEM((CB, D), jnp.float32),
            pltpu.SemaphoreType.DMA,
            pltpu.SemaphoreType.DMA,
        ],
        name="moe_combine",
    )(ys, d_all, wgt)

    return (out.reshape(Bb, Tt, D), jnp.float32(0.0))


# speedup vs baseline: 1.5181x; 1.0833x over previous
import functools

import jax
import jax.numpy as jnp
from jax import lax
from jax.experimental import pallas as pl
from jax.experimental.pallas import tpu as pltpu
from jax.experimental.pallas import tpu_sc as plsc

NE = 8
TOPK = 2
TM = 256
RT = 256
LANES = 128
NC, NS, NW = 2, 16, 32
CH = 256
GB = 32
TPW = 128
CB = 16


def _wid():
    return lax.axis_index("s") * NC + lax.axis_index("c")


_GDN = lax.GatherDimensionNumbers(
    offset_dims=(), collapsed_slice_dims=(0,), start_index_map=(0,))


def _splat(vec, lane):
    idx = jnp.full((16, 1), lane, jnp.int32)
    return lax.gather(vec, idx, _GDN, (1,),
                      mode=lax.GatherScatterMode.PROMISE_IN_BOUNDS)


def _shift_fwd(v, k, lane):
    idx = jnp.maximum(lane - k, 0)[:, None]
    g = lax.gather(v, idx, _GDN, (1,),
                   mode=lax.GatherScatterMode.PROMISE_IN_BOUNDS)
    return jnp.where(lane >= k, g, 0)


def _prefix_sum_incl(v, lane):
    s = v
    for k in (1, 2, 4, 8):
        s = s + _shift_fwd(s, k, lane)
    return s


def _router_body(x_ref, wr_ref, e_ref, w_ref):
    xt = x_ref[...]
    logits = jnp.dot(xt, wr_ref[...], preferred_element_type=jnp.float32)
    lane = lax.broadcasted_iota(jnp.int32, logits.shape, 1)
    logits = jnp.where(lane < NE, logits, -1e30)
    m1 = jnp.max(logits, axis=1, keepdims=True)
    a1 = jnp.min(jnp.where(logits >= m1, lane, LANES), axis=1)
    masked = jnp.where(lane == a1[:, None], -1e30, logits)
    m2 = jnp.max(masked, axis=1, keepdims=True)
    a2 = jnp.min(jnp.where(masked >= m2, lane, LANES), axis=1)
    w1 = 1.0 / (1.0 + jnp.exp(m2[:, 0] - m1[:, 0]))
    e_ref[0, :] = a1
    e_ref[1, :] = a2
    w_ref[0, :] = w1
    w_ref[1, :] = 1.0 - w1


def _gelu(h):
    c = 0.7978845608028654
    return 0.5 * h * (1.0 + jnp.tanh(c * (h + 0.044715 * h * h * h)))


def _ffn_body(te_ref, xs_ref, w1_ref, b1_ref, w2_ref, b2_ref, wv_ref, ys_ref):
    xt = xs_ref[...]
    h = jnp.dot(xt, w1_ref[0], preferred_element_type=jnp.float32) + b1_ref[0]
    h = _gelu(h)
    y = jnp.dot(h, w2_ref[0], preferred_element_type=jnp.float32) + b2_ref[0]
    ys_ref[...] = y * wv_ref[0]


def _plan_count_body(e_hbm, lr_hbm, cnt_hbm, e_v, lr_v, cnt_v):
    w = _wid()
    base = w * CH
    pltpu.sync_copy(e_hbm.at[pl.ds(base, CH)], e_v)
    lane = lax.iota(jnp.int32, 16)
    carries = [jnp.zeros((16,), jnp.int32) for _ in range(NE)]
    for v in range(CH // 16):
        ev = e_v[pl.ds(v * 16, 16)]
        lr = jnp.zeros((16,), jnp.int32)
        for e0 in range(NE):
            m = ev == e0
            incl = _prefix_sum_incl(jnp.where(m, 1, 0), lane)
            lr = jnp.where(m, carries[e0] + incl - 1, lr)
            carries[e0] = carries[e0] + _splat(incl, 15)
        lr_v[pl.ds(v * 16, 16)] = lr
    cnt = jnp.zeros((16,), jnp.int32)
    for e0 in range(NE):
        cnt = jnp.where(lane == e0, carries[e0], cnt)
    cnt_v[...] = cnt
    pltpu.sync_copy(lr_v, lr_hbm.at[pl.ds(base, CH)])
    pltpu.sync_copy(cnt_v, cnt_hbm.at[w])


def _plan_apply_body(N, e_hbm, lr_hbm, cnt_hbm, x_hbm, wgt_hbm,
                     xs_hbm, d_hbm, wvec_hbm, te_hbm,
                     e_v, lr_v, cnt_v, dest_v, tok_v, dlin_v, te_v, w_v,
                     rbuf0, rbuf1, gsem, ssem, wsem):
    w = _wid()
    base = w * CH
    lane = lax.iota(jnp.int32, 16)

    pltpu.sync_copy(cnt_hbm, cnt_v)
    total = jnp.zeros((16,), jnp.int32)
    pre = jnp.zeros((16,), jnp.int32)
    for wp in range(NW):
        row = cnt_v[wp]
        coef = jnp.full((16,), (w > wp).astype(jnp.int32))
        pre = pre + row * coef
        total = total + row
    padded = ((total + (TM - 1)) >> 8) << 8
    ends = _prefix_sum_incl(padded, lane)
    starts = ends - padded
    basev = starts + pre

    pltpu.sync_copy(e_hbm.at[pl.ds(base, CH)], e_v)
    pltpu.sync_copy(lr_hbm.at[pl.ds(base, CH)], lr_v)
    soff = (w >= NW // TOPK).astype(jnp.int32) * N
    for v in range(CH // 16):
        ev = e_v[pl.ds(v * 16, 16)]
        dest = lr_v[pl.ds(v * 16, 16)]
        for e0 in range(NE):
            dest = dest + jnp.where(ev == e0, _splat(basev, e0), 0)
        tok = jnp.full((16,), base - soff + v * 16, jnp.int32) + lane
        vpg = GB // 16
        dest_v[v // vpg, pl.ds((v % vpg) * 16, 16)] = dest
        tok_v[v // vpg, pl.ds((v % vpg) * 16, 16)] = tok
        dlin_v[pl.ds(v * 16, 16)] = dest
    pltpu.sync_copy(dlin_v, d_hbm.at[pl.ds(base, CH)])
    pltpu.sync_copy(wgt_hbm.at[pl.ds(base, CH)], w_v)

    @pl.when(w == 0)
    def _():
        for v in range(3):
            tstart = (jnp.full((16,), v * 16, jnp.int32) + lane) * TM
            te = jnp.zeros((16,), jnp.int32)
            for e0 in range(NE):
                te = te + jnp.where(tstart >= _splat(ends, e0), 1, 0)
            te_v[pl.ds(v * 16, 16)] = te - (te >> 3)
        pltpu.sync_copy(te_v, te_hbm)

    nb = CH // GB
    rbufs = (rbuf0, rbuf1)
    gds = []
    sds = []
    wds = []
    for c in range(nb):
        if c >= 2:
            sds[c - 2].wait()
        gds.append(pltpu.async_copy(x_hbm.at[tok_v.at[c]], rbufs[c % 2], gsem))
        wds.append(pltpu.async_copy(w_v.at[pl.ds(c * GB, GB)],
                                    wvec_hbm.at[dest_v.at[c]], wsem))
        if c >= 1:
            gds[c - 1].wait()
            sds.append(pltpu.async_copy(rbufs[(c - 1) % 2],
                                        xs_hbm.at[dest_v.at[c - 1]], ssem))
    gds[nb - 1].wait()
    sds.append(pltpu.async_copy(rbufs[(nb - 1) % 2],
                                xs_hbm.at[dest_v.at[nb - 1]], ssem))
    sds[nb - 2].wait()
    sds[nb - 1].wait()
    for d in wds:
        d.wait()


def _combine_body(N, D, ys_hbm, d_hbm, out_hbm,
                  d1_v, d2_v, b1a, b2a, b1b, b2b, s1, s2, so):
    w = _wid()
    tb = w * TPW
    pltpu.sync_copy(d_hbm.at[pl.ds(tb, TPW)], d1_v)
    pltpu.sync_copy(d_hbm.at[pl.ds(N + tb, TPW)], d2_v)
    bufs = ((b1a, b2a), (b1b, b2b))
    nb = TPW // CB
    nv = CB * D // 16
    gds = []
    ods = []

    def gissue(c):
        b1, b2 = bufs[c % 2]
        gds.append((
            pltpu.async_copy(ys_hbm.at[d1_v.at[pl.ds(c * CB, CB)]], b1, s1),
            pltpu.async_copy(ys_hbm.at[d2_v.at[pl.ds(c * CB, CB)]], b2, s2),
        ))

    gissue(0)
    for c in range(nb):
        if c >= 1:
            ods[c - 1].wait()
        if c + 1 < nb:
            gissue(c + 1)
        g1, g2 = gds[c]
        g1.wait()
        g2.wait()
        b1, b2 = bufs[c % 2]

        def add_body(i, _):
            r = i >> 6
            cc = (i & 63) * 16
            b1[r, pl.ds(cc, 16)] = b1[r, pl.ds(cc, 16)] + b2[r, pl.ds(cc, 16)]
            return 0

        lax.fori_loop(0, nv, add_body, 0, unroll=8)
        ods.append(pltpu.async_copy(b1, out_hbm.at[pl.ds(tb + c * CB, CB)], so))
    ods[nb - 1].wait()


def kernel(x, Wr, W1, b1, W2, b2):
    Bb, Tt, D = x.shape
    N = Bb * Tt
    DE = W1.shape[2]
    NA = TOPK * N
    NT = (NA + NE * TM) // TM
    NP = NT * TM
    NTP = ((NT + 15) // 16) * 16
    x_flat = x.reshape(N, D)

    wrp = jnp.zeros((D, LANES), jnp.float32).at[:, :NE].set(Wr)
    eidx, wgt = pl.pallas_call(
        _router_body,
        grid=(N // RT,),
        in_specs=[
            pl.BlockSpec((RT, D), lambda i: (i, 0)),
            pl.BlockSpec((D, LANES), lambda i: (0, 0)),
        ],
        out_specs=[
            pl.BlockSpec((TOPK, RT), lambda i: (0, i)),
            pl.BlockSpec((TOPK, RT), lambda i: (0, i)),
        ],
        out_shape=[
            jax.ShapeDtypeStruct((TOPK, N), jnp.int32),
            jax.ShapeDtypeStruct((TOPK, N), jnp.float32),
        ],
    )(x_flat, wrp)
    e_flat = eidx.reshape(NA)

    mesh = plsc.VectorSubcoreMesh(core_axis_name="c", subcore_axis_name="s")

    lr, cnt = pl.kernel(
        _plan_count_body,
        out_type=[
            jax.ShapeDtypeStruct((NA,), jnp.int32),
            jax.ShapeDtypeStruct((NW, 16), jnp.int32),
        ],
        mesh=mesh,
        scratch_types=[
            pltpu.VMEM((CH,), jnp.int32),
            pltpu.VMEM((CH,), jnp.int32),
            pltpu.VMEM((16,), jnp.int32),
        ],
        name="moe_plan_count",
    )(e_flat)

    xs, d_all, wvec, tile_e = pl.kernel(
        functools.partial(_plan_apply_body, N),
        out_type=[
            jax.ShapeDtypeStruct((NP, D), jnp.float32),
            jax.ShapeDtypeStruct((NA,), jnp.int32),
            jax.ShapeDtypeStruct((NP,), jnp.float32),
            jax.ShapeDtypeStruct((NTP,), jnp.int32),
        ],
        mesh=mesh,
        scratch_types=[
            pltpu.VMEM((CH,), jnp.int32),
            pltpu.VMEM((CH,), jnp.int32),
            pltpu.VMEM((NW, 16), jnp.int32),
            pltpu.VMEM((CH // GB, GB), jnp.int32),
            pltpu.VMEM((CH // GB, GB), jnp.int32),
            pltpu.VMEM((CH,), jnp.int32),
            pltpu.VMEM((NTP,), jnp.int32),
            pltpu.VMEM((CH,), jnp.float32),
            pltpu.VMEM((GB, D), jnp.float32),
            pltpu.VMEM((GB, D), jnp.float32),
            pltpu.SemaphoreType.DMA,
            pltpu.SemaphoreType.DMA,
            pltpu.SemaphoreType.DMA,
        ],
        name="moe_plan_apply",
    )(e_flat, lr, cnt, x_flat, wgt.reshape(NA))

    b1r = b1.reshape(NE, 1, DE)
    b2r = b2.reshape(NE, 1, D)
    ys = pl.pallas_call(
        _ffn_body,
        grid_spec=pltpu.PrefetchScalarGridSpec(
            num_scalar_prefetch=1,
            grid=(NT,),
            in_specs=[
                pl.BlockSpec((TM, D), lambda i, te: (i, 0)),
                pl.BlockSpec((1, D, DE), lambda i, te: (te[i], 0, 0)),
                pl.BlockSpec((1, 1, DE), lambda i, te: (te[i], 0, 0)),
                pl.BlockSpec((1, DE, D), lambda i, te: (te[i], 0, 0)),
                pl.BlockSpec((1, 1, D), lambda i, te: (te[i], 0, 0)),
                pl.BlockSpec((1, TM, 1), lambda i, te: (i, 0, 0)),
            ],
            out_specs=pl.BlockSpec((TM, D), lambda i, te: (i, 0)),
        ),
        out_shape=jax.ShapeDtypeStruct((NP, D), jnp.float32),
        compiler_params=pltpu.CompilerParams(
            dimension_semantics=("arbitrary",),
        ),
    )(tile_e[:NT], xs, W1, b1r, W2, b2r, wvec.reshape(NT, TM, 1))

    out = pl.kernel(
        functools.partial(_combine_body, N, D),
        out_type=jax.ShapeDtypeStruct((N, D), jnp.float32),
        mesh=mesh,
        scratch_types=[
            pltpu.VMEM((TPW,), jnp.int32),
            pltpu.VMEM((TPW,), jnp.int32),
            pltpu.VMEM((CB, D), jnp.float32),
            pltpu.VMEM((CB, D), jnp.float32),
            pltpu.VMEM((CB, D), jnp.float32),
            pltpu.VMEM((CB, D), jnp.float32),
            pltpu.SemaphoreType.DMA,
            pltpu.SemaphoreType.DMA,
            pltpu.SemaphoreType.DMA,
        ],
        name="moe_combine",
    )(ys, d_all)

    return (out.reshape(Bb, Tt, D), jnp.float32(0.0))
